# baseline (speedup 1.0000x reference)
import functools

import numpy as np
import jax
import jax.numpy as jnp
from jax.experimental import pallas as pl
from jax.experimental.pallas import tpu as pltpu


def _dma_body(segs, mask_starts, upr, x_ref, mt_ref, o_ref, seg_sems, row_sems):
    copies = []
    for k, (r0, n) in enumerate(segs):
        c = pltpu.make_async_copy(
            x_ref.at[pl.ds(r0, n), :], o_ref.at[pl.ds(r0, n), :],
            seg_sems.at[k])
        c.start()
        copies.append(c)
    for k, r0 in enumerate(mask_starts):
        c = pltpu.make_async_copy(
            mt_ref, o_ref.at[pl.ds(r0, upr), :], row_sems.at[k])
        c.start()
        copies.append(c)
    for c in copies:
        c.wait()


def _masked_positions(S):
    with jax.ensure_compile_time_eval():
        idx_arr = jax.random.choice(
            jax.random.key(42), S, shape=(3,), replace=False)
        return sorted(int(v) for v in np.asarray(idx_arr))


def kernel(x, mask_token):
    B, S, E = x.shape
    idx = _masked_positions(S)

    upr = E // 128
    R = B * S * upr
    flat_rows = sorted(b * S + s for b in range(B) for s in idx)
    segs = []
    prev = 0
    for r in flat_rows:
        r0 = r * upr
        if r0 > prev:
            segs.append((prev, r0 - prev))
        prev = r0 + upr
    if R > prev:
        segs.append((prev, R - prev))
    mask_starts = [r * upr for r in flat_rows]

    xf = x.reshape(R, 128)
    mtf = mask_token.reshape(upr, 128)
    out = pl.pallas_call(
        functools.partial(_dma_body, segs, mask_starts, upr),
        in_specs=[
            pl.BlockSpec(memory_space=pl.ANY),
            pl.BlockSpec(memory_space=pl.ANY),
        ],
        out_specs=pl.BlockSpec(memory_space=pl.ANY),
        out_shape=jax.ShapeDtypeStruct((R, 128), x.dtype),
        scratch_shapes=[
            pltpu.SemaphoreType.DMA((len(segs),)),
            pltpu.SemaphoreType.DMA((len(mask_starts),)),
        ],
    )(xf, mtf)
    return out.reshape(B, S, E)


# speedup vs baseline: 10.7133x; 10.7133x over previous
import functools

import numpy as np
import jax
from jax import lax
import jax.numpy as jnp
from jax.experimental import pallas as pl
from jax.experimental.pallas import tpu as pltpu
from jax.experimental.pallas import tpu_sc as plsc


@functools.lru_cache
def _masked_positions(S):
    with jax.ensure_compile_time_eval():
        idx_arr = jax.random.choice(
            jax.random.key(42), S, shape=(3,), replace=False)
        return sorted(int(v) for v in np.asarray(idx_arr))


def _sc_body(nchunks, chunk, nbuf, upr, mask_starts, rows_per_tile, nc,
             x_ref, mt_ref, o_ref, *scr):
    bufs = scr[:nbuf]
    mtbuf = scr[nbuf]
    sin = scr[nbuf + 1: 2 * nbuf + 1]
    sout = scr[2 * nbuf + 1: 3 * nbuf + 1]
    msem = scr[3 * nbuf + 1]

    wid = lax.axis_index("s") * nc + lax.axis_index("c")
    base = wid * rows_per_tile

    pltpu.sync_copy(mt_ref, mtbuf)

    def start_in(c):
        return pltpu.async_copy(
            x_ref.at[pl.ds(base + c * chunk, chunk), :], bufs[c % nbuf],
            sin[c % nbuf])

    def start_out(c):
        return pltpu.async_copy(
            bufs[c % nbuf], o_ref.at[pl.ds(base + c * chunk, chunk), :],
            sout[c % nbuf])

    in_h = {0: start_in(0)}
    out_h = {}
    for c in range(nchunks):
        in_h.pop(c).wait()
        out_h[c] = start_out(c)
        nxt = c + 1
        if nxt < nchunks:
            if nxt - nbuf in out_h:
                out_h.pop(nxt - nbuf).wait()
            in_h[nxt] = start_in(nxt)
    for c in sorted(out_h):
        out_h.pop(c).wait()

    for r0 in mask_starts:
        @pl.when((r0 >= base) & (r0 < base + rows_per_tile))
        def _():
            pltpu.async_copy(mtbuf, o_ref.at[pl.ds(r0, upr), :], msem).wait()


def kernel(x, mask_token):
    B, S, E = x.shape
    idx = _masked_positions(S)

    upr = E // 128
    R = B * S * upr
    mask_starts = [ (b * S + s) * upr for b in range(B) for s in idx ]

    info = plsc.get_sparse_core_info()
    nc, ns = info.num_cores, info.num_subcores
    nw = nc * ns
    rows_per_tile = R // nw
    chunk = 128
    nbuf = 4
    nchunks = rows_per_tile // chunk

    xf = x.reshape(R, 128)
    mtf = mask_token.reshape(upr, 128)

    f32 = jnp.float32
    sc_kernel = pl.kernel(
        functools.partial(_sc_body, nchunks, chunk, nbuf, upr, mask_starts,
                          rows_per_tile, nc),
        mesh=plsc.VectorSubcoreMesh(core_axis_name="c", subcore_axis_name="s"),
        out_type=jax.ShapeDtypeStruct((R, 128), f32),
        scratch_types=(
            [pltpu.VMEM((chunk, 128), f32) for _ in range(nbuf)]
            + [pltpu.VMEM((upr, 128), f32)]
            + [pltpu.SemaphoreType.DMA for _ in range(2 * nbuf + 1)]
        ),
    )
    out = sc_kernel(xf, mtf)
    return out.reshape(B, S, E)


# speedup vs baseline: 11.6834x; 1.0906x over previous
import functools

import numpy as np
import jax
from jax import lax
import jax.numpy as jnp
from jax.experimental import pallas as pl
from jax.experimental.pallas import tpu as pltpu
from jax.experimental.pallas import tpu_sc as plsc


@functools.lru_cache
def _masked_positions(S):
    with jax.ensure_compile_time_eval():
        idx_arr = jax.random.choice(
            jax.random.key(42), S, shape=(3,), replace=False)
        return tuple(sorted(int(v) for v in np.asarray(idx_arr)))


def _copy_body(x_ref, o_ref):
    o_ref[...] = x_ref[...]


def _sc_scatter_body(mask_starts, upr, nc, o_ref, mt_ref, mtbuf, msem):
    wid = lax.axis_index("s") * nc + lax.axis_index("c")
    for k, r0 in enumerate(mask_starts):
        @pl.when(wid == k)
        def _():
            pltpu.async_copy(mt_ref, mtbuf, msem).wait()
            pltpu.async_copy(mtbuf, o_ref.at[pl.ds(r0, upr), :], msem).wait()


def kernel(x, mask_token):
    B, S, E = x.shape
    idx = _masked_positions(S)

    upr = E // 128
    R = B * S * upr
    mask_starts = [(b * S + s) * upr for b in range(B) for s in idx]

    blk = 512
    xr = x.reshape(B, S, E)
    out = pl.pallas_call(
        _copy_body,
        grid=(S // blk,),
        in_specs=[pl.BlockSpec((B, blk, E), lambda i: (0, i, 0))],
        out_specs=pl.BlockSpec((B, blk, E), lambda i: (0, i, 0)),
        out_shape=jax.ShapeDtypeStruct((B, S, E), x.dtype),
    )(xr)

    info = plsc.get_sparse_core_info()
    nc = info.num_cores
    outf = out.reshape(R, 128)
    mtf = mask_token.reshape(upr, 128)
    f32 = jnp.float32
    sc_scatter = pl.kernel(
        functools.partial(_sc_scatter_body, mask_starts, upr, nc),
        mesh=plsc.VectorSubcoreMesh(core_axis_name="c", subcore_axis_name="s"),
        out_type=(),
        scratch_types=[
            pltpu.VMEM((upr, 128), f32),
            pltpu.SemaphoreType.DMA,
        ],
    )
    oref = jax.new_ref(outf)
    sc_scatter(oref, mtf)
    return oref[...].reshape(B, S, E)
